# baseline (speedup 1.0000x reference)
import functools

import jax
import jax.numpy as jnp
from jax import lax
from jax.experimental import pallas as pl
from jax.experimental.pallas import tpu as pltpu
from jax.experimental.pallas import tpu_sc as plsc

_NNUM = (4096, 16384, 65536, 262144)
_TOTAL = sum(_NNUM)
_D = 128
_NDEPTH = 4
_L = 16

_NC, _NS = 2, 16
_NW = _NC * _NS

_CH_ROWS = 128
_CH = _CH_ROWS * _D
_NCH = _TOTAL // _CH_ROWS
_CPW = _NCH // _NW

_CB = (32, 160, 672)

_DRAIN_CH = 5 * _CH
_NDRAIN = _CPW * _CH // _DRAIN_CH


def _depth_of(c):
    d = jnp.int32(0)
    for b in _CB:
        d = d + (c >= b).astype(jnp.int32)
    return d


@functools.partial(
    pl.kernel,
    out_type=jax.ShapeDtypeStruct((_TOTAL * _D,), jnp.float32),
    mesh=plsc.VectorSubcoreMesh(core_axis_name="c", subcore_axis_name="s"),
    scratch_types=[
        pltpu.VMEM((_NDEPTH * _D,), jnp.float32),
        pltpu.VMEM((_CH,), jnp.float32),
        pltpu.VMEM((_CH,), jnp.float32),
        pltpu.VMEM((_DRAIN_CH,), jnp.float32),
        pltpu.SemaphoreType.DMA,
    ],
)
def _depth_pos_emb(emb_hbm, out_hbm, emb_v, buf_a, buf_b, drain_v, sem):
    wid = lax.axis_index("s") * _NC + lax.axis_index("c")
    pltpu.sync_copy(emb_hbm, emb_v)

    c0 = wid * _CPW
    d_lo = _depth_of(c0)
    d_hi = _depth_of(c0 + _CPW - 1)

    split = jnp.int32(_CPW)
    for b in _CB:
        rel = b - c0
        inside = jnp.logical_and(rel > 0, rel < _CPW)
        split = jnp.where(inside, jnp.minimum(split, rel), split)

    def fill(buf, d):
        row_vecs = [emb_v[pl.ds(d * _D + _L * j, _L)] for j in range(_D // _L)]

        def fill_rows(r, carry):
            for u in range(4):
                base = (4 * r + u) * _D
                for j in range(_D // _L):
                    buf[pl.ds(base + _L * j, _L)] = row_vecs[j]
            return carry

        lax.fori_loop(0, _CH_ROWS // 4, fill_rows, 0)

    def fire_range(lo, hi, buf):
        def body(k, carry):
            pltpu.async_copy(buf, out_hbm.at[pl.ds((c0 + k) * _CH, _CH)], sem)
            return carry

        lax.fori_loop(lo, hi, body, 0)

    fill(buf_a, d_lo)
    fire_range(jnp.int32(0), split, buf_a)
    fill(buf_b, d_hi)
    fire_range(split, jnp.int32(_CPW), buf_b)

    def drain(k, carry):
        pltpu.make_async_copy(out_hbm.at[pl.ds(0, _DRAIN_CH)], drain_v, sem).wait()
        return carry

    lax.fori_loop(0, _NDRAIN, drain, 0)


def kernel(data, depth_emb):
    del data
    out = _depth_pos_emb(depth_emb.reshape(-1))
    return out.reshape(_TOTAL, _D)
